# baseline (speedup 1.0000x reference)
import functools

import jax
import jax.numpy as jnp
from jax import lax
from jax.experimental import pallas as pl
from jax.experimental.pallas import tpu as pltpu
from jax.experimental.pallas import tpu_sc as plsc

N = 20000
NPAD = 20480
NT = 16
PT = NPAD // NT
NV = PT // 16
K = 1000
KPAD = 1024
SENT = 0x7FFFFFFF
SQRT_MAGIC = 0x1FBD1DF5


def _keys_body(c0_ref, c1_ref, c2_ref, o_ref):
    m = jnp.maximum(jnp.maximum(c0_ref[...], c1_ref[...]), c2_ref[...])
    s = 1.0 / (1.0 + jnp.exp(-m))
    o_ref[...] = lax.bitcast_convert_type(s, jnp.int32)


def _make_keys(c0, c1, c2):
    return pl.pallas_call(
        _keys_body,
        out_shape=jax.ShapeDtypeStruct((NPAD // 128, 128), jnp.int32),
    )(c0, c1, c2)


def _lane(vec, i, iota):
    return jnp.sum(jnp.where(iota == i, vec, 0))


def _sc_body(keys_hbm, anch_hbm, bbox_hbm, cls_hbm, dirp_hbm,
             bb_o, sc_o, dir_o, idx_o,
             keys_v, banks, ghist, mrg, sfull, cntv, lk, li,
             stagek, stagei, akey, aidx, bkey, bidx, cntb,
             gidx, ibuf, abuf, obuf, dir_l,
             HIST, DEC, CNT, SURVK, SURVI, GAT, sem):
    t = lax.axis_index("s")
    iota = lax.iota(jnp.int32, 16)
    ones = jnp.ones((16,), jnp.int32)
    zeros = jnp.zeros((16,), jnp.int32)

    pltpu.sync_copy(keys_hbm.at[pl.ds(t * PT, PT)], keys_v)

    def _round(r, carry):
        P, needed = carry
        shift = 24 - 8 * r
        maskc = jnp.where(r == 0, 0,
                          (-1) << jnp.minimum(shift + 8, 31))

        def _zero(i, _):
            banks[pl.ds(i * 16, 16)] = zeros
            return 0
        lax.fori_loop(0, 256, _zero, 0)

        def _hist(v, _):
            u = keys_v[pl.ds(v * 16, 16)]
            m = (u & maskc) == P
            d = (u >> shift) & 0xFF
            plsc.addupdate_scatter(banks, [iota * 256 + d], ones, mask=m)
            return 0
        lax.fori_loop(0, NV, _hist, 0)

        def _merge(c, _):
            def _acc(l, a):
                return a + banks[pl.ds(l * 256 + c * 16, 16)]
            mrg[pl.ds(c * 16, 16)] = lax.fori_loop(0, 16, _acc, zeros)
            return 0
        lax.fori_loop(0, 16, _merge, 0)

        pltpu.sync_copy(mrg, HIST.at[pl.ds(r * 4096 + t * 256, 256)])
        plsc.subcore_barrier()

        @pl.when(t == 0)
        def _scan():
            pltpu.sync_copy(HIST.at[pl.ds(r * 4096, 4096)], ghist)

            def _gsum(c, _):
                def _acc(tt, a):
                    return a + ghist[pl.ds(tt * 256 + c * 16, 16)]
                mrg[pl.ds(c * 16, 16)] = lax.fori_loop(0, 16, _acc, zeros)
                return 0
            lax.fori_loop(0, 16, _gsum, 0)

            def _sfx(c2, R):
                c = 15 - c2
                h = mrg[pl.ds(c * 16, 16)]
                sfx = jnp.flip(jnp.cumsum(jnp.flip(h, 0)), 0) - h
                sfull[pl.ds(c * 16, 16)] = sfx + R
                return R + jnp.sum(h)
            lax.fori_loop(0, 16, _sfx, jnp.int32(0))

            def _cnt(c, a):
                m = sfull[pl.ds(c * 16, 16)] < needed
                return a + jnp.sum(m.astype(jnp.int32))
            ctrue = lax.fori_loop(0, 16, _cnt, jnp.int32(0))
            dstar = jnp.int32(256) - ctrue
            schunk = sfull[pl.ds((dstar >> 4) * 16, 16)]
            needed2 = needed - _lane(schunk, dstar & 15, iota)
            cntv[...] = jnp.where(iota == 0, dstar,
                                  jnp.where(iota == 1, needed2, 0))
            pltpu.sync_copy(cntv, DEC.at[pl.ds(r * 16, 16)])

        plsc.subcore_barrier()
        pltpu.sync_copy(DEC.at[pl.ds(r * 16, 16)], cntv)
        dec = cntv[...]
        dstar = _lane(dec, jnp.int32(0), iota)
        needed = _lane(dec, jnp.int32(1), iota)
        return (P | (dstar << shift), needed)

    T, needed_eq = lax.fori_loop(0, 4, _round,
                                 (jnp.int32(0), jnp.int32(K)))

    def _cnts(v, c):
        u = keys_v[pl.ds(v * 16, 16)]
        cg = jnp.sum((u > T).astype(jnp.int32))
        ce = jnp.sum((u == T).astype(jnp.int32))
        return (c[0] + cg, c[1] + ce)
    cgt, ceq = lax.fori_loop(0, NV, _cnts, (jnp.int32(0), jnp.int32(0)))
    cntv[...] = jnp.where(iota == 0, cgt, jnp.where(iota == 1, ceq, 0))
    pltpu.sync_copy(cntv, CNT.at[pl.ds(t * 16, 16)])
    plsc.subcore_barrier()

    pltpu.sync_copy(CNT, mrg)
    gt_vec = plsc.load_gather(mrg, [iota * 16])
    eq_vec = plsc.load_gather(mrg, [iota * 16 + 1])
    eq_excl = jnp.cumsum(eq_vec) - eq_vec
    sel_eq = jnp.clip(needed_eq - eq_excl, 0, eq_vec)
    cnt_sel = gt_vec + sel_eq
    sel_excl = jnp.cumsum(cnt_sel) - cnt_sel
    my_limit = _lane(sel_eq, t, iota)

    def _fill(i, _):
        lk[pl.ds(i * 16, 16)] = zeros
        li[pl.ds(i * 16, 16)] = zeros + SENT
        return 0
    lax.fori_loop(0, 64, _fill, 0)

    def _compact(v, c):
        csel, ceqr = c
        u = keys_v[pl.ds(v * 16, 16)]
        idxv = t * PT + v * 16 + iota
        gt = u > T
        eq = u == T
        eqc = eq.astype(jnp.int32)
        eq_ex = jnp.cumsum(eqc) - eqc
        sel = gt | (eq & ((ceqr + eq_ex) < my_limit))
        sc = sel.astype(jnp.int32)
        s_ex = jnp.cumsum(sc) - sc
        pos = csel + s_ex
        plsc.store_scatter(lk, [pos], u, mask=sel)
        plsc.store_scatter(li, [pos], idxv, mask=sel)
        return (csel + jnp.sum(sc), ceqr + jnp.sum(eqc))
    lax.fori_loop(0, NV, _compact, (jnp.int32(0), jnp.int32(0)))

    pltpu.sync_copy(lk, SURVK.at[pl.ds(t * KPAD, KPAD)])
    pltpu.sync_copy(li, SURVI.at[pl.ds(t * KPAD, KPAD)])
    plsc.subcore_barrier()

    @pl.when(t == 0)
    def _sort():
        pltpu.sync_copy(SURVK, stagek)
        pltpu.sync_copy(SURVI, stagei)

        def _fa(i, _):
            akey[pl.ds(i * 16, 16)] = zeros
            aidx[pl.ds(i * 16, 16)] = zeros + SENT
            return 0
        lax.fori_loop(0, 64, _fa, 0)

        def _tile(t2, _):
            cnt_t = _lane(cnt_sel, t2, iota)
            base_t = _lane(sel_excl, t2, iota)
            nvv = (cnt_t + 15) >> 4

            def _pull(v2, _):
                u = stagek[pl.ds(t2 * KPAD + v2 * 16, 16)]
                ii = stagei[pl.ds(t2 * KPAD + v2 * 16, 16)]
                m = (v2 * 16 + iota) < cnt_t
                p = base_t + v2 * 16 + iota
                slot = ((p & 63) << 4) | (p >> 6)
                plsc.store_scatter(akey, [slot], u, mask=m)
                plsc.store_scatter(aidx, [slot], ii, mask=m)
                return 0
            lax.fori_loop(0, nvv, _pull, 0)
            return 0
        lax.fori_loop(0, 16, _tile, 0)

        def _one_pass(sk, si, dk, di, shift, last):
            def _zc(i, _):
                cntb[pl.ds(i * 16, 16)] = zeros
                return 0
            lax.fori_loop(0, 16, _zc, 0)

            def _h(v, _):
                u = sk[pl.ds(v * 16, 16)]
                d = 15 - ((u >> shift) & 15)
                plsc.addupdate_scatter(cntb, [d * 16 + iota], ones)
                return 0
            lax.fori_loop(0, 64, _h, 0)

            def _sc(c, R):
                h = cntb[pl.ds(c * 16, 16)]
                cntb[pl.ds(c * 16, 16)] = jnp.cumsum(h) - h + R
                return R + jnp.sum(h)
            lax.fori_loop(0, 16, _sc, jnp.int32(0))

            def _p(v, _):
                u = sk[pl.ds(v * 16, 16)]
                d = 15 - ((u >> shift) & 15)
                bi = d * 16 + iota
                pos = plsc.load_gather(cntb, [bi])
                plsc.store_scatter(cntb, [bi], pos + 1)
                tslot = ((pos & 63) << 4) | (pos >> 6)
                slot = jnp.where(last, pos, tslot)
                plsc.store_scatter(dk, [slot], u)
                ii = si[pl.ds(v * 16, 16)]
                plsc.store_scatter(di, [slot], ii)
                return 0
            lax.fori_loop(0, 64, _p, 0)

        def _dpass(i, _):
            _one_pass(akey, aidx, bkey, bidx, 8 * i, jnp.bool_(False))
            _one_pass(bkey, bidx, akey, aidx, 8 * i + 4, i == 3)
            return 0
        lax.fori_loop(0, 4, _dpass, 0)

        def _g(v, _):
            lk[pl.ds(v * 16, 16)] = jnp.minimum(aidx[pl.ds(v * 16, 16)],
                                                jnp.int32(N - 1))
            return 0
        lax.fori_loop(0, 64, _g, 0)
        pltpu.sync_copy(lk, GAT)
        pltpu.sync_copy(lk, idx_o)

    plsc.subcore_barrier()

    pltpu.sync_copy(GAT.at[pl.ds(t * 64, 64)], gidx)

    def _ifill(g, _):
        gv = gidx[pl.ds(g * 16, 16)]

        def _ik(k, _):
            stride = jnp.where(k < 14, 7, jnp.where(k < 17, 3, 2))
            col = jnp.where(k < 7, k,
                            jnp.where(k < 14, k - 7,
                                      jnp.where(k < 17, k - 14, k - 17)))
            ibuf[pl.ds(k * 64 + g * 16, 16)] = gv * stride + col
            return 0
        lax.fori_loop(0, 19, _ik, 0)
        return 0
    lax.fori_loop(0, 4, _ifill, 0)

    tabs = (anch_hbm,) * 7 + (bbox_hbm,) * 7 + (cls_hbm,) * 3 + (dirp_hbm,) * 2
    descs = [
        pltpu.async_copy(tab.at[ibuf.at[pl.ds(k * 64, 64)]],
                         abuf.at[pl.ds(k * 64, 64)], sem)
        for k, tab in enumerate(tabs)
    ]
    for d in descs:
        d.wait()

    def _dec(g, _):
        def _col(k):
            return abuf[pl.ds(k * 64 + g * 16, 16)]

        xa = _col(0); ya = _col(1); za = _col(2)
        wa = _col(3); la = _col(4); ha = _col(5); ra = _col(6)
        xt = _col(7); yt = _col(8); zt = _col(9)
        wt = _col(10); lt = _col(11); ht = _col(12); rt = _col(13)

        za2 = za + ha * 0.5
        a2 = la * la + wa * wa
        yi = SQRT_MAGIC + (plsc.bitcast(a2, jnp.int32) >> 1)
        y = plsc.bitcast(yi, jnp.float32)
        y = 0.5 * (y + a2 / y)
        y = 0.5 * (y + a2 / y)
        y = 0.5 * (y + a2 / y)
        diag = y
        xg = xt * diag + xa
        yg = yt * diag + ya
        zg = zt * ha + za2
        lg = jnp.exp(lt) * la
        wg = jnp.exp(wt) * wa
        hg = jnp.exp(ht) * ha
        rg = rt + ra
        zg = zg - hg * 0.5

        for c, val in enumerate((xg, yg, zg, wg, lg, hg, rg)):
            obuf[pl.ds(c * 64 + g * 16, 16)] = val
        for c in range(3):
            x = _col(14 + c)
            obuf[pl.ds((7 + c) * 64 + g * 16, 16)] = 1.0 / (1.0 + jnp.exp(-x))
        d0 = _col(17)
        d1 = _col(18)
        dir_l[pl.ds(g * 16, 16)] = jnp.where(d1 > d0, 1, 0).astype(jnp.int32)
        return 0
    lax.fori_loop(0, 4, _dec, 0)

    for c in range(7):
        pltpu.sync_copy(obuf.at[pl.ds(c * 64, 64)],
                        bb_o.at[c, pl.ds(t * 64, 64)])
    for c in range(3):
        pltpu.sync_copy(obuf.at[pl.ds((7 + c) * 64, 64)],
                        sc_o.at[c, pl.ds(t * 64, 64)])
    pltpu.sync_copy(dir_l, dir_o.at[pl.ds(t * 64, 64)])


@functools.partial(jax.jit, static_argnums=())
def _sc_topk(keys, anchors, bbox_pred, cls_score, dir_cls_pred):
    mesh = plsc.VectorSubcoreMesh(core_axis_name="c", subcore_axis_name="s",
                                  num_cores=1)
    f = pl.kernel(
        _sc_body,
        out_type=[
            jax.ShapeDtypeStruct((7, KPAD), jnp.float32),
            jax.ShapeDtypeStruct((3, KPAD), jnp.float32),
            jax.ShapeDtypeStruct((KPAD,), jnp.int32),
            jax.ShapeDtypeStruct((KPAD,), jnp.int32),
        ],
        mesh=mesh,
        compiler_params=pltpu.CompilerParams(needs_layout_passes=False,
                                             use_tc_tiling_on_sc=False),
        scratch_types=[
            pltpu.VMEM((PT,), jnp.int32),
            pltpu.VMEM((4096,), jnp.int32),
            pltpu.VMEM((4096,), jnp.int32),
            pltpu.VMEM((256,), jnp.int32),
            pltpu.VMEM((256,), jnp.int32),
            pltpu.VMEM((16,), jnp.int32),
            pltpu.VMEM((KPAD,), jnp.int32),
            pltpu.VMEM((KPAD,), jnp.int32),
            pltpu.VMEM((16 * KPAD,), jnp.int32),
            pltpu.VMEM((16 * KPAD,), jnp.int32),
            pltpu.VMEM((KPAD,), jnp.int32),
            pltpu.VMEM((KPAD,), jnp.int32),
            pltpu.VMEM((KPAD,), jnp.int32),
            pltpu.VMEM((KPAD,), jnp.int32),
            pltpu.VMEM((256,), jnp.int32),
            pltpu.VMEM((64,), jnp.int32),
            pltpu.VMEM((19 * 64,), jnp.int32),
            pltpu.VMEM((19 * 64,), jnp.float32),
            pltpu.VMEM((10 * 64,), jnp.float32),
            pltpu.VMEM((64,), jnp.int32),
            pltpu.VMEM_SHARED((4 * 4096,), jnp.int32),
            pltpu.VMEM_SHARED((64,), jnp.int32),
            pltpu.VMEM_SHARED((256,), jnp.int32),
            pltpu.VMEM_SHARED((16 * KPAD,), jnp.int32),
            pltpu.VMEM_SHARED((16 * KPAD,), jnp.int32),
            pltpu.VMEM_SHARED((KPAD,), jnp.int32),
            pltpu.SemaphoreType.DMA,
        ],
    )
    return f(keys, anchors, bbox_pred, cls_score, dir_cls_pred)


def kernel(cls_score, bbox_pred, dir_cls_pred, anchors):
    pad = jnp.full((NPAD - N,), -200.0, jnp.float32)
    cs = [jnp.concatenate([cls_score[:, i], pad]).reshape(NPAD // 128, 128)
          for i in range(3)]
    keys = _make_keys(*cs).reshape(NPAD)
    bb, sc, dr, _ = _sc_topk(keys, anchors.reshape(-1), bbox_pred.reshape(-1),
                             cls_score.reshape(-1), dir_cls_pred.reshape(-1))
    return (bb.T[:K], sc.T[:K], dr[:K])


# speedup vs baseline: 1.5313x; 1.5313x over previous
import functools

import jax
import jax.numpy as jnp
from jax import lax
from jax.experimental import pallas as pl
from jax.experimental.pallas import tpu as pltpu
from jax.experimental.pallas import tpu_sc as plsc

N = 20000
NPAD = 20480
NT = 16
PT = NPAD // NT
NV = PT // 16
K = 1000
KPAD = 1024
SENT = 0x7FFFFFFF
SQRT_MAGIC = 0x1FBD1DF5


def _keys_body(c0_ref, c1_ref, c2_ref, o_ref):
    m = jnp.maximum(jnp.maximum(c0_ref[...], c1_ref[...]), c2_ref[...])
    s = 1.0 / (1.0 + jnp.exp(-m))
    o_ref[...] = lax.bitcast_convert_type(s, jnp.int32)


def _make_keys(c0, c1, c2):
    return pl.pallas_call(
        _keys_body,
        out_shape=jax.ShapeDtypeStruct((NPAD // 128, 128), jnp.int32),
    )(c0, c1, c2)


def _lane(vec, i, iota):
    return jnp.sum(jnp.where(iota == i, vec, 0))


def _sc_body(keys_hbm, flat_hbm,
             bb_o, sc_o, dir_o, idx_o,
             keys_v, banks, ghist, mrg, sfull, cntv, lk, li,
             stagek, stagei, akey, aidx, bkey, bidx, cntb,
             gidx, ibuf, abuf, obuf, dir_l,
             HIST, DEC, CNT, SURVK, SURVI, GAT, sem):
    t = lax.axis_index("s")
    iota = lax.iota(jnp.int32, 16)
    ones = jnp.ones((16,), jnp.int32)
    zeros = jnp.zeros((16,), jnp.int32)

    pltpu.sync_copy(keys_hbm.at[pl.ds(t * PT, PT)], keys_v)

    def _round(r, carry):
        P, needed = carry
        shift = 24 - 8 * r
        maskc = jnp.where(r == 0, 0,
                          (-1) << jnp.minimum(shift + 8, 31))

        def _zero(i, _):
            banks[pl.ds(i * 16, 16)] = zeros
            return 0
        lax.fori_loop(0, 256, _zero, 0)

        def _hist(v, _):
            u = keys_v[pl.ds(v * 16, 16)]
            m = (u & maskc) == P
            d = (u >> shift) & 0xFF
            plsc.addupdate_scatter(banks, [iota * 256 + d], ones, mask=m)
            return 0
        lax.fori_loop(0, NV, _hist, 0)

        def _merge(c, _):
            def _acc(l, a):
                return a + banks[pl.ds(l * 256 + c * 16, 16)]
            mrg[pl.ds(c * 16, 16)] = lax.fori_loop(0, 16, _acc, zeros)
            return 0
        lax.fori_loop(0, 16, _merge, 0)

        pltpu.sync_copy(mrg, HIST.at[pl.ds(r * 4096 + t * 256, 256)])
        plsc.subcore_barrier()

        @pl.when(t == 0)
        def _scan():
            pltpu.sync_copy(HIST.at[pl.ds(r * 4096, 4096)], ghist)

            def _gsum(c, _):
                def _acc(tt, a):
                    return a + ghist[pl.ds(tt * 256 + c * 16, 16)]
                mrg[pl.ds(c * 16, 16)] = lax.fori_loop(0, 16, _acc, zeros)
                return 0
            lax.fori_loop(0, 16, _gsum, 0)

            def _sfx(c2, R):
                c = 15 - c2
                h = mrg[pl.ds(c * 16, 16)]
                sfx = jnp.flip(jnp.cumsum(jnp.flip(h, 0)), 0) - h
                sfull[pl.ds(c * 16, 16)] = sfx + R
                return R + jnp.sum(h)
            lax.fori_loop(0, 16, _sfx, jnp.int32(0))

            def _cnt(c, a):
                m = sfull[pl.ds(c * 16, 16)] < needed
                return a + jnp.sum(m.astype(jnp.int32))
            ctrue = lax.fori_loop(0, 16, _cnt, jnp.int32(0))
            dstar = jnp.int32(256) - ctrue
            schunk = sfull[pl.ds((dstar >> 4) * 16, 16)]
            needed2 = needed - _lane(schunk, dstar & 15, iota)
            cntv[...] = jnp.where(iota == 0, dstar,
                                  jnp.where(iota == 1, needed2, 0))
            pltpu.sync_copy(cntv, DEC.at[pl.ds(r * 16, 16)])

        plsc.subcore_barrier()
        pltpu.sync_copy(DEC.at[pl.ds(r * 16, 16)], cntv)
        dec = cntv[...]
        dstar = _lane(dec, jnp.int32(0), iota)
        needed = _lane(dec, jnp.int32(1), iota)
        return (P | (dstar << shift), needed)

    T, needed_eq = lax.fori_loop(0, 4, _round,
                                 (jnp.int32(0), jnp.int32(K)))

    def _cnts(v, c):
        u = keys_v[pl.ds(v * 16, 16)]
        cg = jnp.sum((u > T).astype(jnp.int32))
        ce = jnp.sum((u == T).astype(jnp.int32))
        return (c[0] + cg, c[1] + ce)
    cgt, ceq = lax.fori_loop(0, NV, _cnts, (jnp.int32(0), jnp.int32(0)))
    cntv[...] = jnp.where(iota == 0, cgt, jnp.where(iota == 1, ceq, 0))
    pltpu.sync_copy(cntv, CNT.at[pl.ds(t * 16, 16)])
    plsc.subcore_barrier()

    pltpu.sync_copy(CNT, mrg)
    gt_vec = plsc.load_gather(mrg, [iota * 16])
    eq_vec = plsc.load_gather(mrg, [iota * 16 + 1])
    eq_excl = jnp.cumsum(eq_vec) - eq_vec
    sel_eq = jnp.clip(needed_eq - eq_excl, 0, eq_vec)
    cnt_sel = gt_vec + sel_eq
    sel_excl = jnp.cumsum(cnt_sel) - cnt_sel
    my_limit = _lane(sel_eq, t, iota)

    def _fill(i, _):
        lk[pl.ds(i * 16, 16)] = zeros
        li[pl.ds(i * 16, 16)] = zeros + SENT
        return 0
    lax.fori_loop(0, 64, _fill, 0)

    def _compact(v, c):
        csel, ceqr = c
        u = keys_v[pl.ds(v * 16, 16)]
        idxv = t * PT + v * 16 + iota
        gt = u > T
        eq = u == T
        eqc = eq.astype(jnp.int32)
        eq_ex = jnp.cumsum(eqc) - eqc
        sel = gt | (eq & ((ceqr + eq_ex) < my_limit))
        sc = sel.astype(jnp.int32)
        s_ex = jnp.cumsum(sc) - sc
        pos = csel + s_ex
        plsc.store_scatter(lk, [pos], u, mask=sel)
        plsc.store_scatter(li, [pos], idxv, mask=sel)
        return (csel + jnp.sum(sc), ceqr + jnp.sum(eqc))
    lax.fori_loop(0, NV, _compact, (jnp.int32(0), jnp.int32(0)))

    pltpu.sync_copy(lk, SURVK.at[pl.ds(t * KPAD, KPAD)])
    pltpu.sync_copy(li, SURVI.at[pl.ds(t * KPAD, KPAD)])
    plsc.subcore_barrier()

    @pl.when(t == 0)
    def _sort():
        pltpu.sync_copy(SURVK, stagek)
        pltpu.sync_copy(SURVI, stagei)

        def _fa(i, _):
            akey[pl.ds(i * 16, 16)] = zeros
            aidx[pl.ds(i * 16, 16)] = zeros + SENT
            return 0
        lax.fori_loop(0, 64, _fa, 0)

        def _tile(t2, _):
            cnt_t = _lane(cnt_sel, t2, iota)
            base_t = _lane(sel_excl, t2, iota)
            nvv = (cnt_t + 15) >> 4

            def _pull(v2, _):
                u = stagek[pl.ds(t2 * KPAD + v2 * 16, 16)]
                ii = stagei[pl.ds(t2 * KPAD + v2 * 16, 16)]
                m = (v2 * 16 + iota) < cnt_t
                p = base_t + v2 * 16 + iota
                slot = ((p & 63) << 4) | (p >> 6)
                plsc.store_scatter(akey, [slot], u, mask=m)
                plsc.store_scatter(aidx, [slot], ii, mask=m)
                return 0
            lax.fori_loop(0, nvv, _pull, 0)
            return 0
        lax.fori_loop(0, 16, _tile, 0)

        def _one_pass(sk, si, dk, di, shift, last):
            def _zc(i, _):
                cntb[pl.ds(i * 16, 16)] = zeros
                return 0
            lax.fori_loop(0, 16, _zc, 0)

            def _h(v, _):
                u = sk[pl.ds(v * 16, 16)]
                d = 15 - ((u >> shift) & 15)
                plsc.addupdate_scatter(cntb, [d * 16 + iota], ones)
                return 0
            lax.fori_loop(0, 64, _h, 0)

            def _sc(c, R):
                h = cntb[pl.ds(c * 16, 16)]
                cntb[pl.ds(c * 16, 16)] = jnp.cumsum(h) - h + R
                return R + jnp.sum(h)
            lax.fori_loop(0, 16, _sc, jnp.int32(0))

            def _p(v, _):
                u = sk[pl.ds(v * 16, 16)]
                d = 15 - ((u >> shift) & 15)
                bi = d * 16 + iota
                pos = plsc.load_gather(cntb, [bi])
                plsc.store_scatter(cntb, [bi], pos + 1)
                tslot = ((pos & 63) << 4) | (pos >> 6)
                slot = jnp.where(last, pos, tslot)
                plsc.store_scatter(dk, [slot], u)
                ii = si[pl.ds(v * 16, 16)]
                plsc.store_scatter(di, [slot], ii)
                return 0
            lax.fori_loop(0, 64, _p, 0)

        def _dpass(i, _):
            _one_pass(akey, aidx, bkey, bidx, 8 * i, jnp.bool_(False))
            _one_pass(bkey, bidx, akey, aidx, 8 * i + 4, i == 3)
            return 0
        lax.fori_loop(0, 4, _dpass, 0)

        def _g(v, _):
            lk[pl.ds(v * 16, 16)] = jnp.minimum(aidx[pl.ds(v * 16, 16)],
                                                jnp.int32(N - 1))
            return 0
        lax.fori_loop(0, 64, _g, 0)
        pltpu.sync_copy(lk, GAT)
        pltpu.sync_copy(lk, idx_o)

    plsc.subcore_barrier()

    pltpu.sync_copy(GAT.at[pl.ds(t * 64, 64)], gidx)

    def _ifill(g, _):
        gv = gidx[pl.ds(g * 16, 16)]

        def _ik(k, _):
            ibuf[pl.ds(k * 64 + g * 16, 16)] = gv + k * NPAD
            return 0
        lax.fori_loop(0, 19, _ik, 0)
        return 0
    lax.fori_loop(0, 4, _ifill, 0)

    descs = [
        pltpu.async_copy(flat_hbm.at[ibuf.at[pl.ds(k * 64, 64)]],
                         abuf.at[pl.ds(k * 64, 64)], sem)
        for k in range(19)
    ]
    for d in descs:
        d.wait()

    def _dec(g, _):
        def _col(k):
            return abuf[pl.ds(k * 64 + g * 16, 16)]

        xa = _col(0); ya = _col(1); za = _col(2)
        wa = _col(3); la = _col(4); ha = _col(5); ra = _col(6)
        xt = _col(7); yt = _col(8); zt = _col(9)
        wt = _col(10); lt = _col(11); ht = _col(12); rt = _col(13)

        za2 = za + ha * 0.5
        a2 = la * la + wa * wa
        yi = SQRT_MAGIC + (plsc.bitcast(a2, jnp.int32) >> 1)
        y = plsc.bitcast(yi, jnp.float32)
        y = 0.5 * (y + a2 / y)
        y = 0.5 * (y + a2 / y)
        y = 0.5 * (y + a2 / y)
        diag = y
        xg = xt * diag + xa
        yg = yt * diag + ya
        zg = zt * ha + za2
        lg = jnp.exp(lt) * la
        wg = jnp.exp(wt) * wa
        hg = jnp.exp(ht) * ha
        rg = rt + ra
        zg = zg - hg * 0.5

        for c, val in enumerate((xg, yg, zg, wg, lg, hg, rg)):
            obuf[pl.ds(c * 64 + g * 16, 16)] = val
        for c in range(3):
            x = _col(14 + c)
            obuf[pl.ds((7 + c) * 64 + g * 16, 16)] = 1.0 / (1.0 + jnp.exp(-x))
        d0 = _col(17)
        d1 = _col(18)
        dir_l[pl.ds(g * 16, 16)] = jnp.where(d1 > d0, 1, 0).astype(jnp.int32)
        return 0
    lax.fori_loop(0, 4, _dec, 0)

    for c in range(7):
        pltpu.sync_copy(obuf.at[pl.ds(c * 64, 64)],
                        bb_o.at[c, pl.ds(t * 64, 64)])
    for c in range(3):
        pltpu.sync_copy(obuf.at[pl.ds((7 + c) * 64, 64)],
                        sc_o.at[c, pl.ds(t * 64, 64)])
    pltpu.sync_copy(dir_l, dir_o.at[pl.ds(t * 64, 64)])


@functools.partial(jax.jit, static_argnums=())
def _sc_topk(keys, flat_tab):
    mesh = plsc.VectorSubcoreMesh(core_axis_name="c", subcore_axis_name="s",
                                  num_cores=1)
    f = pl.kernel(
        _sc_body,
        out_type=[
            jax.ShapeDtypeStruct((7, KPAD), jnp.float32),
            jax.ShapeDtypeStruct((3, KPAD), jnp.float32),
            jax.ShapeDtypeStruct((KPAD,), jnp.int32),
            jax.ShapeDtypeStruct((KPAD,), jnp.int32),
        ],
        mesh=mesh,
        compiler_params=pltpu.CompilerParams(needs_layout_passes=False,
                                             use_tc_tiling_on_sc=False),
        scratch_types=[
            pltpu.VMEM((PT,), jnp.int32),
            pltpu.VMEM((4096,), jnp.int32),
            pltpu.VMEM((4096,), jnp.int32),
            pltpu.VMEM((256,), jnp.int32),
            pltpu.VMEM((256,), jnp.int32),
            pltpu.VMEM((16,), jnp.int32),
            pltpu.VMEM((KPAD,), jnp.int32),
            pltpu.VMEM((KPAD,), jnp.int32),
            pltpu.VMEM((16 * KPAD,), jnp.int32),
            pltpu.VMEM((16 * KPAD,), jnp.int32),
            pltpu.VMEM((KPAD,), jnp.int32),
            pltpu.VMEM((KPAD,), jnp.int32),
            pltpu.VMEM((KPAD,), jnp.int32),
            pltpu.VMEM((KPAD,), jnp.int32),
            pltpu.VMEM((256,), jnp.int32),
            pltpu.VMEM((64,), jnp.int32),
            pltpu.VMEM((19 * 64,), jnp.int32),
            pltpu.VMEM((19 * 64,), jnp.float32),
            pltpu.VMEM((10 * 64,), jnp.float32),
            pltpu.VMEM((64,), jnp.int32),
            pltpu.VMEM_SHARED((4 * 4096,), jnp.int32),
            pltpu.VMEM_SHARED((64,), jnp.int32),
            pltpu.VMEM_SHARED((256,), jnp.int32),
            pltpu.VMEM_SHARED((16 * KPAD,), jnp.int32),
            pltpu.VMEM_SHARED((16 * KPAD,), jnp.int32),
            pltpu.VMEM_SHARED((KPAD,), jnp.int32),
            pltpu.SemaphoreType.DMA,
        ],
    )
    return f(keys, flat_tab)


def kernel(cls_score, bbox_pred, dir_cls_pred, anchors):
    pad = jnp.full((NPAD - N,), -200.0, jnp.float32)
    cs = [jnp.concatenate([cls_score[:, i], pad]).reshape(NPAD // 128, 128)
          for i in range(3)]
    keys = _make_keys(*cs).reshape(NPAD)
    zpad = jnp.zeros((NPAD - N,), jnp.float32)
    colsrc = ([anchors[:, c] for c in range(7)]
              + [bbox_pred[:, c] for c in range(7)]
              + [cls_score[:, c] for c in range(3)]
              + [dir_cls_pred[:, c] for c in range(2)])
    flat_tab = jnp.stack(
        [jnp.concatenate([col, zpad]) for col in colsrc]
    ).reshape(19 * NPAD // 128, 128).reshape(19 * NPAD)
    bb, sc, dr, _ = _sc_topk(keys, flat_tab)
    return (bb.T[:K], sc.T[:K], dr[:K])


# speedup vs baseline: 1.8556x; 1.2117x over previous
import functools

import jax
import jax.numpy as jnp
from jax import lax
from jax.experimental import pallas as pl
from jax.experimental.pallas import tpu as pltpu
from jax.experimental.pallas import tpu_sc as plsc

N = 20000
NPAD = 20480
NT = 16
PT = NPAD // NT
NV = PT // 16
K = 1000
KPAD = 1024
SENT = 0x7FFFFFFF
SQRT_MAGIC = 0x1FBD1DF5


def _keys_body(c0_ref, c1_ref, c2_ref, o_ref):
    m = jnp.maximum(jnp.maximum(c0_ref[...], c1_ref[...]), c2_ref[...])
    s = 1.0 / (1.0 + jnp.exp(-m))
    o_ref[...] = lax.bitcast_convert_type(s, jnp.int32)


def _make_keys(c0, c1, c2):
    return pl.pallas_call(
        _keys_body,
        out_shape=jax.ShapeDtypeStruct((NPAD // 128, 128), jnp.int32),
    )(c0, c1, c2)


def _lane(vec, i, iota):
    return jnp.sum(jnp.where(iota == i, vec, 0))


def _sc_body(keys_hbm, *rest):
    tabs = rest[:19]
    (bb_o, sc_o, dir_o, idx_o,
     keys_v, banks, ghist, mrg, sfull, cntv, lk, li,
     stagek, stagei, akey, aidx, bkey, bidx, cntb,
     gidx, abuf, obuf, dir_l,
     HIST, DEC, CNT, SURVK, SURVI, GAT, sem) = rest[19:]
    t = lax.axis_index("s")
    iota = lax.iota(jnp.int32, 16)
    ones = jnp.ones((16,), jnp.int32)
    zeros = jnp.zeros((16,), jnp.int32)

    pltpu.sync_copy(keys_hbm.at[pl.ds(t * PT, PT)], keys_v)

    def _round(r, carry):
        P, needed = carry
        shift = 24 - 8 * r
        maskc = jnp.where(r == 0, 0,
                          (-1) << jnp.minimum(shift + 8, 31))

        def _zero(i, _):
            banks[pl.ds(i * 16, 16)] = zeros
            return 0
        lax.fori_loop(0, 256, _zero, 0)

        def _hist(v, _):
            u = keys_v[pl.ds(v * 16, 16)]
            m = (u & maskc) == P
            d = (u >> shift) & 0xFF
            plsc.addupdate_scatter(banks, [iota * 256 + d], ones, mask=m)
            return 0
        lax.fori_loop(0, NV, _hist, 0)

        def _merge(c, _):
            def _acc(l, a):
                return a + banks[pl.ds(l * 256 + c * 16, 16)]
            mrg[pl.ds(c * 16, 16)] = lax.fori_loop(0, 16, _acc, zeros)
            return 0
        lax.fori_loop(0, 16, _merge, 0)

        pltpu.sync_copy(mrg, HIST.at[pl.ds(r * 4096 + t * 256, 256)])
        plsc.subcore_barrier()

        @pl.when(t == 0)
        def _scan():
            pltpu.sync_copy(HIST.at[pl.ds(r * 4096, 4096)], ghist)

            def _gsum(c, _):
                def _acc(tt, a):
                    return a + ghist[pl.ds(tt * 256 + c * 16, 16)]
                mrg[pl.ds(c * 16, 16)] = lax.fori_loop(0, 16, _acc, zeros)
                return 0
            lax.fori_loop(0, 16, _gsum, 0)

            def _sfx(c2, R):
                c = 15 - c2
                h = mrg[pl.ds(c * 16, 16)]
                sfx = jnp.flip(jnp.cumsum(jnp.flip(h, 0)), 0) - h
                sfull[pl.ds(c * 16, 16)] = sfx + R
                return R + jnp.sum(h)
            lax.fori_loop(0, 16, _sfx, jnp.int32(0))

            def _cnt(c, a):
                m = sfull[pl.ds(c * 16, 16)] < needed
                return a + jnp.sum(m.astype(jnp.int32))
            ctrue = lax.fori_loop(0, 16, _cnt, jnp.int32(0))
            dstar = jnp.int32(256) - ctrue
            schunk = sfull[pl.ds((dstar >> 4) * 16, 16)]
            needed2 = needed - _lane(schunk, dstar & 15, iota)
            cntv[...] = jnp.where(iota == 0, dstar,
                                  jnp.where(iota == 1, needed2, 0))
            pltpu.sync_copy(cntv, DEC.at[pl.ds(r * 16, 16)])

        plsc.subcore_barrier()
        pltpu.sync_copy(DEC.at[pl.ds(r * 16, 16)], cntv)
        dec = cntv[...]
        dstar = _lane(dec, jnp.int32(0), iota)
        needed = _lane(dec, jnp.int32(1), iota)
        return (P | (dstar << shift), needed)

    T, needed_eq = lax.fori_loop(0, 4, _round,
                                 (jnp.int32(0), jnp.int32(K)))

    def _cnts(v, c):
        u = keys_v[pl.ds(v * 16, 16)]
        cg = jnp.sum((u > T).astype(jnp.int32))
        ce = jnp.sum((u == T).astype(jnp.int32))
        return (c[0] + cg, c[1] + ce)
    cgt, ceq = lax.fori_loop(0, NV, _cnts, (jnp.int32(0), jnp.int32(0)))
    cntv[...] = jnp.where(iota == 0, cgt, jnp.where(iota == 1, ceq, 0))
    pltpu.sync_copy(cntv, CNT.at[pl.ds(t * 16, 16)])
    plsc.subcore_barrier()

    pltpu.sync_copy(CNT, mrg)
    gt_vec = plsc.load_gather(mrg, [iota * 16])
    eq_vec = plsc.load_gather(mrg, [iota * 16 + 1])
    eq_excl = jnp.cumsum(eq_vec) - eq_vec
    sel_eq = jnp.clip(needed_eq - eq_excl, 0, eq_vec)
    cnt_sel = gt_vec + sel_eq
    sel_excl = jnp.cumsum(cnt_sel) - cnt_sel
    my_limit = _lane(sel_eq, t, iota)

    def _fill(i, _):
        lk[pl.ds(i * 16, 16)] = zeros
        li[pl.ds(i * 16, 16)] = zeros + SENT
        return 0
    lax.fori_loop(0, 64, _fill, 0)

    def _compact(v, c):
        csel, ceqr = c
        u = keys_v[pl.ds(v * 16, 16)]
        idxv = t * PT + v * 16 + iota
        gt = u > T
        eq = u == T
        eqc = eq.astype(jnp.int32)
        eq_ex = jnp.cumsum(eqc) - eqc
        sel = gt | (eq & ((ceqr + eq_ex) < my_limit))
        sc = sel.astype(jnp.int32)
        s_ex = jnp.cumsum(sc) - sc
        pos = csel + s_ex
        plsc.store_scatter(lk, [pos], u, mask=sel)
        plsc.store_scatter(li, [pos], idxv, mask=sel)
        return (csel + jnp.sum(sc), ceqr + jnp.sum(eqc))
    lax.fori_loop(0, NV, _compact, (jnp.int32(0), jnp.int32(0)))

    pltpu.sync_copy(lk, SURVK.at[pl.ds(t * KPAD, KPAD)])
    pltpu.sync_copy(li, SURVI.at[pl.ds(t * KPAD, KPAD)])
    plsc.subcore_barrier()

    @pl.when(t == 0)
    def _sort():
        pltpu.sync_copy(SURVK, stagek)
        pltpu.sync_copy(SURVI, stagei)

        def _fa(i, _):
            akey[pl.ds(i * 16, 16)] = zeros
            aidx[pl.ds(i * 16, 16)] = zeros + SENT
            return 0
        lax.fori_loop(0, 64, _fa, 0)

        def _tile(t2, _):
            cnt_t = _lane(cnt_sel, t2, iota)
            base_t = _lane(sel_excl, t2, iota)
            nvv = (cnt_t + 15) >> 4

            def _pull(v2, _):
                u = stagek[pl.ds(t2 * KPAD + v2 * 16, 16)]
                ii = stagei[pl.ds(t2 * KPAD + v2 * 16, 16)]
                m = (v2 * 16 + iota) < cnt_t
                p = base_t + v2 * 16 + iota
                slot = ((p & 63) << 4) | (p >> 6)
                plsc.store_scatter(akey, [slot], u, mask=m)
                plsc.store_scatter(aidx, [slot], ii, mask=m)
                return 0
            lax.fori_loop(0, nvv, _pull, 0)
            return 0
        lax.fori_loop(0, 16, _tile, 0)

        def _one_pass(sk, si, dk, di, shift, last):
            def _zc(i, _):
                cntb[pl.ds(i * 16, 16)] = zeros
                return 0
            lax.fori_loop(0, 16, _zc, 0)

            def _h(v, _):
                u = sk[pl.ds(v * 16, 16)]
                d = 15 - ((u >> shift) & 15)
                plsc.addupdate_scatter(cntb, [d * 16 + iota], ones)
                return 0
            lax.fori_loop(0, 64, _h, 0)

            def _sc(c, R):
                h = cntb[pl.ds(c * 16, 16)]
                cntb[pl.ds(c * 16, 16)] = jnp.cumsum(h) - h + R
                return R + jnp.sum(h)
            lax.fori_loop(0, 16, _sc, jnp.int32(0))

            def _p(v, _):
                u = sk[pl.ds(v * 16, 16)]
                d = 15 - ((u >> shift) & 15)
                bi = d * 16 + iota
                pos = plsc.load_gather(cntb, [bi])
                plsc.store_scatter(cntb, [bi], pos + 1)
                tslot = ((pos & 63) << 4) | (pos >> 6)
                slot = jnp.where(last, pos, tslot)
                plsc.store_scatter(dk, [slot], u)
                ii = si[pl.ds(v * 16, 16)]
                plsc.store_scatter(di, [slot], ii)
                return 0
            lax.fori_loop(0, 64, _p, 0)

        def _dpass(i, _):
            _one_pass(akey, aidx, bkey, bidx, 8 * i, jnp.bool_(False))
            _one_pass(bkey, bidx, akey, aidx, 8 * i + 4, i == 3)
            return 0
        lax.fori_loop(0, 4, _dpass, 0)

        def _g(v, _):
            lk[pl.ds(v * 16, 16)] = jnp.minimum(aidx[pl.ds(v * 16, 16)],
                                                jnp.int32(N - 1))
            return 0
        lax.fori_loop(0, 64, _g, 0)
        pltpu.sync_copy(lk, GAT)
        pltpu.sync_copy(lk, idx_o)

    plsc.subcore_barrier()

    pltpu.sync_copy(GAT.at[pl.ds(t * 64, 64)], gidx)
    descs = [
        pltpu.async_copy(tabs[k].at[gidx], abuf.at[pl.ds(k * 64, 64)], sem)
        for k in range(19)
    ]
    for d in descs:
        d.wait()

    def _dec(g, _):
        def _col(k):
            return abuf[pl.ds(k * 64 + g * 16, 16)]

        xa = _col(0); ya = _col(1); za = _col(2)
        wa = _col(3); la = _col(4); ha = _col(5); ra = _col(6)
        xt = _col(7); yt = _col(8); zt = _col(9)
        wt = _col(10); lt = _col(11); ht = _col(12); rt = _col(13)

        za2 = za + ha * 0.5
        a2 = la * la + wa * wa
        yi = SQRT_MAGIC + (plsc.bitcast(a2, jnp.int32) >> 1)
        y = plsc.bitcast(yi, jnp.float32)
        y = 0.5 * (y + a2 / y)
        y = 0.5 * (y + a2 / y)
        y = 0.5 * (y + a2 / y)
        diag = y
        xg = xt * diag + xa
        yg = yt * diag + ya
        zg = zt * ha + za2
        lg = jnp.exp(lt) * la
        wg = jnp.exp(wt) * wa
        hg = jnp.exp(ht) * ha
        rg = rt + ra
        zg = zg - hg * 0.5

        for c, val in enumerate((xg, yg, zg, wg, lg, hg, rg)):
            obuf[pl.ds(c * 64 + g * 16, 16)] = val
        for c in range(3):
            x = _col(14 + c)
            obuf[pl.ds((7 + c) * 64 + g * 16, 16)] = 1.0 / (1.0 + jnp.exp(-x))
        d0 = _col(17)
        d1 = _col(18)
        dir_l[pl.ds(g * 16, 16)] = jnp.where(d1 > d0, 1, 0).astype(jnp.int32)
        return 0
    lax.fori_loop(0, 4, _dec, 0)

    for c in range(7):
        pltpu.sync_copy(obuf.at[pl.ds(c * 64, 64)],
                        bb_o.at[c, pl.ds(t * 64, 64)])
    for c in range(3):
        pltpu.sync_copy(obuf.at[pl.ds((7 + c) * 64, 64)],
                        sc_o.at[c, pl.ds(t * 64, 64)])
    pltpu.sync_copy(dir_l, dir_o.at[pl.ds(t * 64, 64)])


@functools.partial(jax.jit, static_argnums=())
def _sc_topk(keys, *cols):
    mesh = plsc.VectorSubcoreMesh(core_axis_name="c", subcore_axis_name="s",
                                  num_cores=1)
    f = pl.kernel(
        _sc_body,
        out_type=[
            jax.ShapeDtypeStruct((7, KPAD), jnp.float32),
            jax.ShapeDtypeStruct((3, KPAD), jnp.float32),
            jax.ShapeDtypeStruct((KPAD,), jnp.int32),
            jax.ShapeDtypeStruct((KPAD,), jnp.int32),
        ],
        mesh=mesh,
        compiler_params=pltpu.CompilerParams(needs_layout_passes=False,
                                             use_tc_tiling_on_sc=False),
        scratch_types=[
            pltpu.VMEM((PT,), jnp.int32),
            pltpu.VMEM((4096,), jnp.int32),
            pltpu.VMEM((4096,), jnp.int32),
            pltpu.VMEM((256,), jnp.int32),
            pltpu.VMEM((256,), jnp.int32),
            pltpu.VMEM((16,), jnp.int32),
            pltpu.VMEM((KPAD,), jnp.int32),
            pltpu.VMEM((KPAD,), jnp.int32),
            pltpu.VMEM((16 * KPAD,), jnp.int32),
            pltpu.VMEM((16 * KPAD,), jnp.int32),
            pltpu.VMEM((KPAD,), jnp.int32),
            pltpu.VMEM((KPAD,), jnp.int32),
            pltpu.VMEM((KPAD,), jnp.int32),
            pltpu.VMEM((KPAD,), jnp.int32),
            pltpu.VMEM((256,), jnp.int32),
            pltpu.VMEM((64,), jnp.int32),
            pltpu.VMEM((19 * 64,), jnp.float32),
            pltpu.VMEM((10 * 64,), jnp.float32),
            pltpu.VMEM((64,), jnp.int32),
            pltpu.VMEM_SHARED((4 * 4096,), jnp.int32),
            pltpu.VMEM_SHARED((64,), jnp.int32),
            pltpu.VMEM_SHARED((256,), jnp.int32),
            pltpu.VMEM_SHARED((16 * KPAD,), jnp.int32),
            pltpu.VMEM_SHARED((16 * KPAD,), jnp.int32),
            pltpu.VMEM_SHARED((KPAD,), jnp.int32),
            pltpu.SemaphoreType.DMA,
        ],
    )
    return f(keys, *cols)


def kernel(cls_score, bbox_pred, dir_cls_pred, anchors):
    pad = jnp.full((NPAD - N,), -200.0, jnp.float32)
    cs = [jnp.concatenate([cls_score[:, i], pad]).reshape(NPAD // 128, 128)
          for i in range(3)]
    keys = _make_keys(*cs).reshape(NPAD)
    colsrc = ([anchors[:, c] for c in range(7)]
              + [bbox_pred[:, c] for c in range(7)]
              + [cls_score[:, c] for c in range(3)]
              + [dir_cls_pred[:, c] for c in range(2)])
    bb, sc, dr, _ = _sc_topk(keys, *colsrc)
    return (bb.T[:K], sc.T[:K], dr[:K])


# speedup vs baseline: 1.9033x; 1.0257x over previous
import functools

import jax
import jax.numpy as jnp
from jax import lax
from jax.experimental import pallas as pl
from jax.experimental.pallas import tpu as pltpu
from jax.experimental.pallas import tpu_sc as plsc

N = 20000
NPAD = 20480
NT = 16
PT = NPAD // NT
NV = PT // 16
K = 1000
KPAD = 1024
SENT = 0x7FFFFFFF
SQRT_MAGIC = 0x1FBD1DF5


def _keys_body(c0_ref, c1_ref, c2_ref, o_ref):
    m = jnp.maximum(jnp.maximum(c0_ref[...], c1_ref[...]), c2_ref[...])
    s = 1.0 / (1.0 + jnp.exp(-m))
    o_ref[...] = lax.bitcast_convert_type(s, jnp.int32)


def _make_keys(c0, c1, c2):
    return pl.pallas_call(
        _keys_body,
        out_shape=jax.ShapeDtypeStruct((NPAD // 128, 128), jnp.int32),
    )(c0, c1, c2)


def _lane(vec, i, iota):
    return jnp.sum(jnp.where(iota == i, vec, 0))


def _sc_body(keys_hbm, *rest):
    tabs = rest[:19]
    (bb_o, sc_o, dir_o, idx_o,
     keys_v, banks, ghist, mrg, sfull, cntv, lk, li,
     stagek, stagei, akey, aidx, bkey, bidx, cntb,
     gidx, abuf, obuf, dir_l,
     HIST, DEC, CNT, SURVK, SURVI, GAT, sem) = rest[19:]
    t = lax.axis_index("s")
    iota = lax.iota(jnp.int32, 16)
    ones = jnp.ones((16,), jnp.int32)
    zeros = jnp.zeros((16,), jnp.int32)

    pltpu.sync_copy(keys_hbm.at[pl.ds(t * PT, PT)], keys_v)

    def _round(r, carry):
        P, needed = carry
        shift = 24 - 8 * r
        maskc = jnp.where(r == 0, 0,
                          (-1) << jnp.minimum(shift + 8, 31))

        def _zero(i, _):
            banks[pl.ds(i * 16, 16)] = zeros
            return 0
        lax.fori_loop(0, 256, _zero, 0)

        def _hist(v, _):
            u = keys_v[pl.ds(v * 16, 16)]
            m = (u & maskc) == P
            d = (u >> shift) & 0xFF
            plsc.addupdate_scatter(banks, [iota * 256 + d], ones, mask=m)
            return 0
        lax.fori_loop(0, NV, _hist, 0)

        def _merge(c, _):
            def _acc(l, a):
                return a + banks[pl.ds(l * 256 + c * 16, 16)]
            mrg[pl.ds(c * 16, 16)] = lax.fori_loop(0, 16, _acc, zeros)
            return 0
        lax.fori_loop(0, 16, _merge, 0)

        pltpu.sync_copy(mrg, HIST.at[pl.ds(r * 4096 + t * 256, 256)])
        plsc.subcore_barrier()

        @pl.when(t == 0)
        def _scan():
            pltpu.sync_copy(HIST.at[pl.ds(r * 4096, 4096)], ghist)

            def _gsum(c, _):
                def _acc(tt, a):
                    return a + ghist[pl.ds(tt * 256 + c * 16, 16)]
                mrg[pl.ds(c * 16, 16)] = lax.fori_loop(0, 16, _acc, zeros)
                return 0
            lax.fori_loop(0, 16, _gsum, 0)

            def _sfx(c2, R):
                c = 15 - c2
                h = mrg[pl.ds(c * 16, 16)]
                sfx = jnp.flip(jnp.cumsum(jnp.flip(h, 0)), 0) - h
                sfull[pl.ds(c * 16, 16)] = sfx + R
                return R + jnp.sum(h)
            lax.fori_loop(0, 16, _sfx, jnp.int32(0))

            def _cnt(c, a):
                m = sfull[pl.ds(c * 16, 16)] < needed
                return a + jnp.sum(m.astype(jnp.int32))
            ctrue = lax.fori_loop(0, 16, _cnt, jnp.int32(0))
            dstar = jnp.int32(256) - ctrue
            schunk = sfull[pl.ds((dstar >> 4) * 16, 16)]
            needed2 = needed - _lane(schunk, dstar & 15, iota)
            cntv[...] = jnp.where(iota == 0, dstar,
                                  jnp.where(iota == 1, needed2, 0))
            pltpu.sync_copy(cntv, DEC.at[pl.ds(r * 16, 16)])

        plsc.subcore_barrier()
        pltpu.sync_copy(DEC.at[pl.ds(r * 16, 16)], cntv)
        dec = cntv[...]
        dstar = _lane(dec, jnp.int32(0), iota)
        needed = _lane(dec, jnp.int32(1), iota)
        return (P | (dstar << shift), needed)

    T, needed_eq = lax.fori_loop(0, 4, _round,
                                 (jnp.int32(0), jnp.int32(K)))

    def _cnts(v, c):
        u = keys_v[pl.ds(v * 16, 16)]
        cg = jnp.sum((u > T).astype(jnp.int32))
        ce = jnp.sum((u == T).astype(jnp.int32))
        return (c[0] + cg, c[1] + ce)
    cgt, ceq = lax.fori_loop(0, NV, _cnts, (jnp.int32(0), jnp.int32(0)))
    cntv[...] = jnp.where(iota == 0, cgt, jnp.where(iota == 1, ceq, 0))
    pltpu.sync_copy(cntv, CNT.at[pl.ds(t * 16, 16)])
    plsc.subcore_barrier()

    pltpu.sync_copy(CNT, mrg)
    gt_vec = plsc.load_gather(mrg, [iota * 16])
    eq_vec = plsc.load_gather(mrg, [iota * 16 + 1])
    eq_excl = jnp.cumsum(eq_vec) - eq_vec
    sel_eq = jnp.clip(needed_eq - eq_excl, 0, eq_vec)
    cnt_sel = gt_vec + sel_eq
    sel_excl = jnp.cumsum(cnt_sel) - cnt_sel
    my_limit = _lane(sel_eq, t, iota)

    def _fill(i, _):
        lk[pl.ds(i * 16, 16)] = zeros
        li[pl.ds(i * 16, 16)] = zeros + SENT
        return 0
    lax.fori_loop(0, 64, _fill, 0)

    def _compact(v, c):
        csel, ceqr = c
        u = keys_v[pl.ds(v * 16, 16)]
        idxv = t * PT + v * 16 + iota
        gt = u > T
        eq = u == T
        eqc = eq.astype(jnp.int32)
        eq_ex = jnp.cumsum(eqc) - eqc
        sel = gt | (eq & ((ceqr + eq_ex) < my_limit))
        sc = sel.astype(jnp.int32)
        s_ex = jnp.cumsum(sc) - sc
        pos = csel + s_ex
        plsc.store_scatter(lk, [pos], u, mask=sel)
        plsc.store_scatter(li, [pos], idxv, mask=sel)
        return (csel + jnp.sum(sc), ceqr + jnp.sum(eqc))
    lax.fori_loop(0, NV, _compact, (jnp.int32(0), jnp.int32(0)))

    pltpu.sync_copy(lk, SURVK.at[pl.ds(t * KPAD, KPAD)])
    pltpu.sync_copy(li, SURVI.at[pl.ds(t * KPAD, KPAD)])
    plsc.subcore_barrier()

    @pl.when(t == 0)
    def _sort():
        pltpu.sync_copy(SURVK, stagek)
        pltpu.sync_copy(SURVI, stagei)

        def _fa(i, _):
            akey[pl.ds(i * 16, 16)] = zeros
            aidx[pl.ds(i * 16, 16)] = zeros + SENT
            return 0
        lax.fori_loop(0, 64, _fa, 0)

        def _tile(t2, _):
            cnt_t = _lane(cnt_sel, t2, iota)
            base_t = _lane(sel_excl, t2, iota)
            nvv = (cnt_t + 15) >> 4

            def _pull(v2, _):
                u = stagek[pl.ds(t2 * KPAD + v2 * 16, 16)]
                ii = stagei[pl.ds(t2 * KPAD + v2 * 16, 16)]
                m = (v2 * 16 + iota) < cnt_t
                p = base_t + v2 * 16 + iota
                slot = ((p & 63) << 4) | (p >> 6)
                plsc.store_scatter(akey, [slot], u, mask=m)
                plsc.store_scatter(aidx, [slot], ii, mask=m)
                return 0
            lax.fori_loop(0, nvv, _pull, 0)
            return 0
        lax.fori_loop(0, 16, _tile, 0)

        def _one_pass(sk, si, dk, di, shift, last):
            def _zc(i, _):
                for j in range(4):
                    cntb[pl.ds((i * 4 + j) * 16, 16)] = zeros
                return 0
            lax.fori_loop(0, 16, _zc, 0)

            def _h(v, _):
                for j in range(4):
                    u = sk[pl.ds((v * 4 + j) * 16, 16)]
                    d = 63 - ((u >> shift) & 63)
                    plsc.addupdate_scatter(cntb, [d * 16 + iota], ones)
                return 0
            lax.fori_loop(0, 16, _h, 0)

            def _sc(c, R):
                h = cntb[pl.ds(c * 16, 16)]
                cntb[pl.ds(c * 16, 16)] = jnp.cumsum(h) - h + R
                return R + jnp.sum(h)
            lax.fori_loop(0, 64, _sc, jnp.int32(0))

            def _p(v, _):
                for j in range(4):
                    u = sk[pl.ds((v * 4 + j) * 16, 16)]
                    d = 63 - ((u >> shift) & 63)
                    bi = d * 16 + iota
                    pos = plsc.load_gather(cntb, [bi])
                    plsc.store_scatter(cntb, [bi], pos + 1)
                    tslot = ((pos & 63) << 4) | (pos >> 6)
                    slot = jnp.where(last, pos, tslot)
                    plsc.store_scatter(dk, [slot], u)
                    ii = si[pl.ds((v * 4 + j) * 16, 16)]
                    plsc.store_scatter(di, [slot], ii)
                return 0
            lax.fori_loop(0, 16, _p, 0)

        def _dpass(i, _):
            _one_pass(akey, aidx, bkey, bidx, 12 * i, jnp.bool_(False))
            _one_pass(bkey, bidx, akey, aidx, 12 * i + 6, i == 2)
            return 0
        lax.fori_loop(0, 3, _dpass, 0)

        def _g(v, _):
            lk[pl.ds(v * 16, 16)] = jnp.minimum(aidx[pl.ds(v * 16, 16)],
                                                jnp.int32(N - 1))
            return 0
        lax.fori_loop(0, 64, _g, 0)
        pltpu.sync_copy(lk, GAT)
        pltpu.sync_copy(lk, idx_o)

    plsc.subcore_barrier()

    pltpu.sync_copy(GAT.at[pl.ds(t * 64, 64)], gidx)
    descs = [
        pltpu.async_copy(tabs[k].at[gidx], abuf.at[pl.ds(k * 64, 64)], sem)
        for k in range(19)
    ]
    for d in descs:
        d.wait()

    def _dec(g, _):
        def _col(k):
            return abuf[pl.ds(k * 64 + g * 16, 16)]

        xa = _col(0); ya = _col(1); za = _col(2)
        wa = _col(3); la = _col(4); ha = _col(5); ra = _col(6)
        xt = _col(7); yt = _col(8); zt = _col(9)
        wt = _col(10); lt = _col(11); ht = _col(12); rt = _col(13)

        za2 = za + ha * 0.5
        a2 = la * la + wa * wa
        yi = SQRT_MAGIC + (plsc.bitcast(a2, jnp.int32) >> 1)
        y = plsc.bitcast(yi, jnp.float32)
        y = 0.5 * (y + a2 / y)
        y = 0.5 * (y + a2 / y)
        y = 0.5 * (y + a2 / y)
        diag = y
        xg = xt * diag + xa
        yg = yt * diag + ya
        zg = zt * ha + za2
        lg = jnp.exp(lt) * la
        wg = jnp.exp(wt) * wa
        hg = jnp.exp(ht) * ha
        rg = rt + ra
        zg = zg - hg * 0.5

        for c, val in enumerate((xg, yg, zg, wg, lg, hg, rg)):
            obuf[pl.ds(c * 64 + g * 16, 16)] = val
        for c in range(3):
            x = _col(14 + c)
            obuf[pl.ds((7 + c) * 64 + g * 16, 16)] = 1.0 / (1.0 + jnp.exp(-x))
        d0 = _col(17)
        d1 = _col(18)
        dir_l[pl.ds(g * 16, 16)] = jnp.where(d1 > d0, 1, 0).astype(jnp.int32)
        return 0
    lax.fori_loop(0, 4, _dec, 0)

    for c in range(7):
        pltpu.sync_copy(obuf.at[pl.ds(c * 64, 64)],
                        bb_o.at[c, pl.ds(t * 64, 64)])
    for c in range(3):
        pltpu.sync_copy(obuf.at[pl.ds((7 + c) * 64, 64)],
                        sc_o.at[c, pl.ds(t * 64, 64)])
    pltpu.sync_copy(dir_l, dir_o.at[pl.ds(t * 64, 64)])


@functools.partial(jax.jit, static_argnums=())
def _sc_topk(keys, *cols):
    mesh = plsc.VectorSubcoreMesh(core_axis_name="c", subcore_axis_name="s",
                                  num_cores=1)
    f = pl.kernel(
        _sc_body,
        out_type=[
            jax.ShapeDtypeStruct((7, KPAD), jnp.float32),
            jax.ShapeDtypeStruct((3, KPAD), jnp.float32),
            jax.ShapeDtypeStruct((KPAD,), jnp.int32),
            jax.ShapeDtypeStruct((KPAD,), jnp.int32),
        ],
        mesh=mesh,
        compiler_params=pltpu.CompilerParams(needs_layout_passes=False,
                                             use_tc_tiling_on_sc=False),
        scratch_types=[
            pltpu.VMEM((PT,), jnp.int32),
            pltpu.VMEM((4096,), jnp.int32),
            pltpu.VMEM((4096,), jnp.int32),
            pltpu.VMEM((256,), jnp.int32),
            pltpu.VMEM((256,), jnp.int32),
            pltpu.VMEM((16,), jnp.int32),
            pltpu.VMEM((KPAD,), jnp.int32),
            pltpu.VMEM((KPAD,), jnp.int32),
            pltpu.VMEM((16 * KPAD,), jnp.int32),
            pltpu.VMEM((16 * KPAD,), jnp.int32),
            pltpu.VMEM((KPAD,), jnp.int32),
            pltpu.VMEM((KPAD,), jnp.int32),
            pltpu.VMEM((KPAD,), jnp.int32),
            pltpu.VMEM((KPAD,), jnp.int32),
            pltpu.VMEM((1024,), jnp.int32),
            pltpu.VMEM((64,), jnp.int32),
            pltpu.VMEM((19 * 64,), jnp.float32),
            pltpu.VMEM((10 * 64,), jnp.float32),
            pltpu.VMEM((64,), jnp.int32),
            pltpu.VMEM_SHARED((4 * 4096,), jnp.int32),
            pltpu.VMEM_SHARED((64,), jnp.int32),
            pltpu.VMEM_SHARED((256,), jnp.int32),
            pltpu.VMEM_SHARED((16 * KPAD,), jnp.int32),
            pltpu.VMEM_SHARED((16 * KPAD,), jnp.int32),
            pltpu.VMEM_SHARED((KPAD,), jnp.int32),
            pltpu.SemaphoreType.DMA,
        ],
    )
    return f(keys, *cols)


def kernel(cls_score, bbox_pred, dir_cls_pred, anchors):
    pad = jnp.full((NPAD - N,), -200.0, jnp.float32)
    cs = [jnp.concatenate([cls_score[:, i], pad]).reshape(NPAD // 128, 128)
          for i in range(3)]
    keys = _make_keys(*cs).reshape(NPAD)
    colsrc = ([anchors[:, c] for c in range(7)]
              + [bbox_pred[:, c] for c in range(7)]
              + [cls_score[:, c] for c in range(3)]
              + [dir_cls_pred[:, c] for c in range(2)])
    bb, sc, dr, _ = _sc_topk(keys, *colsrc)
    return (bb.T[:K], sc.T[:K], dr[:K])
